# baseline (speedup 1.0000x reference)
import functools
import jax
import jax.numpy as jnp
from jax import lax
from jax.experimental import pallas as pl
from jax.experimental.pallas import tpu as pltpu
from jax.experimental.pallas import tpu_sc as plsc

D = 64
B = 1024
L = 200
VOCAB = 100000

LP = 256
NCHUNK = LP // 16
HIST_N = 100096
NWORKERS = 32
ROWS_PER_W = B // NWORKERS
CN = 2048
CROWS = 104
PADN = CROWS * CN


def _sc_counts(src, dst):
    mesh = plsc.VectorSubcoreMesh(core_axis_name="c", subcore_axis_name="s")
    f32 = jnp.float32

    @functools.partial(
        pl.kernel,
        mesh=mesh,
        compiler_params=pltpu.CompilerParams(needs_layout_passes=False),
        out_type=[jax.ShapeDtypeStruct((PADN,), f32) for _ in range(4)],
        scratch_types=[
            pltpu.VMEM((HIST_N,), f32),
            pltpu.VMEM((LP,), jnp.int32),
            pltpu.VMEM((LP,), jnp.int32),
            pltpu.VMEM((LP,), f32),
            pltpu.VMEM((LP,), f32),
            pltpu.VMEM((LP,), f32),
            pltpu.VMEM((LP,), f32),
        ],
    )
    def sc_kernel(src_hbm, dst_hbm, css_hbm, csd_hbm, cds_hbm, cdd_hbm,
                  hist, srow, drow, css, csd, cds, cdd):
        cid = lax.axis_index("c")
        sid = lax.axis_index("s")
        wid = sid * 2 + cid
        base = wid * ROWS_PER_W

        zero16 = jnp.zeros((16,), f32)
        one16 = jnp.ones((16,), f32)

        def zinit(i, carry):
            off = pl.multiple_of(i * 256, 256)
            for j in range(16):
                hist[pl.ds(off + j * 16, 16)] = zero16
            return carry
        lax.fori_loop(0, HIST_N // 256, zinit, 0)

        def load_row(hbm, buf, b):
            pltpu.sync_copy(hbm.at[b, pl.ds(0, 128)], buf.at[pl.ds(0, 128)])
            pltpu.sync_copy(hbm.at[b, pl.ds(128, 128)], buf.at[pl.ds(128, 128)])

        def hist_add(idbuf):
            for c in range(NCHUNK):
                plsc.addupdate_scatter(hist, [idbuf[pl.ds(c * 16, 16)]], one16)

        def hist_clear(idbuf):
            for c in range(NCHUNK):
                plsc.store_scatter(hist, [idbuf[pl.ds(c * 16, 16)]], zero16)

        def gather_counts(idbuf, out_buf):
            for c in range(NCHUNK):
                idx = idbuf[pl.ds(c * 16, 16)]
                g = plsc.load_gather(hist, [idx])
                out_buf[pl.ds(c * 16, 16)] = jnp.where(idx == 0, 0.0, g)

        def row_body(i, carry):
            b = base + i
            load_row(src_hbm, srow, b)
            load_row(dst_hbm, drow, b)

            hist_add(srow)
            gather_counts(srow, css)
            gather_counts(drow, cds)
            hist_clear(srow)

            hist_add(drow)
            gather_counts(srow, csd)
            gather_counts(drow, cdd)
            hist_clear(drow)

            off = b * L
            pltpu.sync_copy(css.at[pl.ds(0, L)], css_hbm.at[pl.ds(off, L)])
            pltpu.sync_copy(csd.at[pl.ds(0, L)], csd_hbm.at[pl.ds(off, L)])
            pltpu.sync_copy(cds.at[pl.ds(0, L)], cds_hbm.at[pl.ds(off, L)])
            pltpu.sync_copy(cdd.at[pl.ds(0, L)], cdd_hbm.at[pl.ds(off, L)])
            return carry

        lax.fori_loop(0, ROWS_PER_W, row_body, 0)

    srcp, dstp = _tc_pad(src, dst)
    return sc_kernel(srcp, dstp)


def _tc_pad(src, dst):
    PB = 256

    def body(s_ref, d_ref, so_ref, do_ref):
        zeros = jnp.zeros((PB, LP - L), jnp.int32)
        so_ref[:, :L] = s_ref[...]
        so_ref[:, L:] = zeros
        do_ref[:, :L] = d_ref[...]
        do_ref[:, L:] = zeros

    in_spec = pl.BlockSpec((PB, L), lambda i: (i, 0))
    out_spec = pl.BlockSpec((PB, LP), lambda i: (i, 0))
    return pl.pallas_call(
        body,
        grid=(B // PB,),
        in_specs=[in_spec, in_spec],
        out_specs=[out_spec, out_spec],
        out_shape=[jax.ShapeDtypeStruct((B, LP), jnp.int32)] * 2,
    )(src, dst)


def _tc_encode(css, csd, cds, cdd, w1c, b1c, w2, b2c):
    grid = ((B * L) // CN,)

    def body(css_ref, csd_ref, cds_ref, cdd_ref, w1_ref, b1_ref, w2_ref,
             b2_ref, osrc_ref, odst_ref):
        j = pl.program_id(0) % 8
        w1 = w1_ref[...]
        b1 = b1_ref[...]
        w2 = w2_ref[...]
        b2 = b2_ref[...]

        def enc(c_ref0, c_ref1, out_ref):
            c0 = c_ref0[pl.ds(j, 1), :]
            c1 = c_ref1[pl.ds(j, 1), :]
            h = (jnp.maximum(c0 * w1 + b1, 0.0)
                 + jnp.maximum(c1 * w1 + b1, 0.0))
            out_t = jnp.dot(w2, h, preferred_element_type=jnp.float32) \
                + 2.0 * b2
            out_ref[...] = out_t.T

        enc(css_ref, csd_ref, osrc_ref)
        enc(cds_ref, cdd_ref, odst_ref)

    cnt_spec = pl.BlockSpec((8, CN), lambda i: (i // 8, 0))
    wcol_spec = pl.BlockSpec((D, 1), lambda i: (0, 0))
    w2_spec = pl.BlockSpec((D, D), lambda i: (0, 0))
    out_spec = pl.BlockSpec((CN, D), lambda i: (i, 0))

    return pl.pallas_call(
        body,
        grid=grid,
        in_specs=[cnt_spec, cnt_spec, cnt_spec, cnt_spec,
                  wcol_spec, wcol_spec, w2_spec, wcol_spec],
        out_specs=[out_spec, out_spec],
        out_shape=[jax.ShapeDtypeStruct((B * L, D), jnp.float32)] * 2,
    )(css, csd, cds, cdd, w1c, b1c, w2, b2c)


@jax.jit
def kernel(src_padded_nodes_neighbor_ids, dst_padded_nodes_neighbor_ids,
           W1, b1, W2, b2):
    src = src_padded_nodes_neighbor_ids
    dst = dst_padded_nodes_neighbor_ids

    c_ss, c_sd, c_ds, c_dd = _sc_counts(src, dst)

    b1c = b1.reshape(D, 1)
    b2c = b2.reshape(D, 1)

    out_src, out_dst = _tc_encode(
        c_ss.reshape(CROWS, CN), c_sd.reshape(CROWS, CN),
        c_ds.reshape(CROWS, CN), c_dd.reshape(CROWS, CN),
        W1, b1c, W2, b2c)

    return out_src.reshape(B, L, D), out_dst.reshape(B, L, D)


# speedup vs baseline: 1.8289x; 1.8289x over previous
import functools
import jax
import jax.numpy as jnp
from jax import lax
from jax.experimental import pallas as pl
from jax.experimental.pallas import tpu as pltpu
from jax.experimental.pallas import tpu_sc as plsc

D = 64
B = 1024
L = 200
VOCAB = 100000

LP = 256
NCHUNK = LP // 16
HIST_N = 100096
NWORKERS = 32
ROWS_PER_W = B // NWORKERS


def _sc_counts(src, dst):
    mesh = plsc.VectorSubcoreMesh(core_axis_name="c", subcore_axis_name="s")
    f32 = jnp.float32

    @functools.partial(
        pl.kernel,
        mesh=mesh,
        compiler_params=pltpu.CompilerParams(needs_layout_passes=False),
        out_type=[jax.ShapeDtypeStruct((B * L,), f32) for _ in range(4)],
        scratch_types=[
            pltpu.VMEM((HIST_N,), f32),
            pltpu.VMEM((LP,), jnp.int32),
            pltpu.VMEM((LP,), jnp.int32),
            pltpu.VMEM((LP,), f32),
            pltpu.VMEM((LP,), f32),
            pltpu.VMEM((LP,), f32),
            pltpu.VMEM((LP,), f32),
        ],
    )
    def sc_kernel(src_hbm, dst_hbm, css_hbm, csd_hbm, cds_hbm, cdd_hbm,
                  hist, srow, drow, css, csd, cds, cdd):
        cid = lax.axis_index("c")
        sid = lax.axis_index("s")
        wid = sid * 2 + cid
        base = wid * ROWS_PER_W

        zero16 = jnp.zeros((16,), f32)
        one16 = jnp.ones((16,), f32)

        def zinit(i, carry):
            off = pl.multiple_of(i * 256, 256)
            for j in range(16):
                hist[pl.ds(off + j * 16, 16)] = zero16
            return carry
        lax.fori_loop(0, HIST_N // 256, zinit, 0)

        def load_row(hbm, buf, b):
            pltpu.sync_copy(hbm.at[b, pl.ds(0, 128)], buf.at[pl.ds(0, 128)])
            pltpu.sync_copy(hbm.at[b, pl.ds(128, 128)], buf.at[pl.ds(128, 128)])

        def hist_add(idbuf):
            for c in range(NCHUNK):
                plsc.addupdate_scatter(hist, [idbuf[pl.ds(c * 16, 16)]], one16)

        def hist_clear(idbuf):
            for c in range(NCHUNK):
                plsc.store_scatter(hist, [idbuf[pl.ds(c * 16, 16)]], zero16)

        def gather_counts(idbuf, out_buf):
            for c in range(NCHUNK):
                idx = idbuf[pl.ds(c * 16, 16)]
                g = plsc.load_gather(hist, [idx])
                out_buf[pl.ds(c * 16, 16)] = jnp.where(idx == 0, 0.0, g)

        def row_body(i, carry):
            b = base + i
            load_row(src_hbm, srow, b)
            load_row(dst_hbm, drow, b)

            hist_add(srow)
            gather_counts(srow, css)
            gather_counts(drow, cds)
            hist_clear(srow)

            hist_add(drow)
            gather_counts(srow, csd)
            gather_counts(drow, cdd)
            hist_clear(drow)

            off = b * L
            pltpu.sync_copy(css.at[pl.ds(0, L)], css_hbm.at[pl.ds(off, L)])
            pltpu.sync_copy(csd.at[pl.ds(0, L)], csd_hbm.at[pl.ds(off, L)])
            pltpu.sync_copy(cds.at[pl.ds(0, L)], cds_hbm.at[pl.ds(off, L)])
            pltpu.sync_copy(cdd.at[pl.ds(0, L)], cdd_hbm.at[pl.ds(off, L)])
            return carry

        lax.fori_loop(0, ROWS_PER_W, row_body, 0)

    srcp, dstp = _tc_pad(src, dst)
    return sc_kernel(srcp, dstp)


def _tc_pad(src, dst):
    PB = 256

    def body(s_ref, d_ref, so_ref, do_ref):
        zeros = jnp.zeros((PB, LP - L), jnp.int32)
        so_ref[:, :L] = s_ref[...]
        so_ref[:, L:] = zeros
        do_ref[:, :L] = d_ref[...]
        do_ref[:, L:] = zeros

    in_spec = pl.BlockSpec((PB, L), lambda i: (i, 0))
    out_spec = pl.BlockSpec((PB, LP), lambda i: (i, 0))
    return pl.pallas_call(
        body,
        grid=(B // PB,),
        in_specs=[in_spec, in_spec],
        out_specs=[out_spec, out_spec],
        out_shape=[jax.ShapeDtypeStruct((B, LP), jnp.int32)] * 2,
    )(src, dst)


def _tc_encode(css, csd, cds, cdd, w1c, b1c, w2, b2c):
    LB = 8
    grid = (L // LB,)

    def body(css_ref, csd_ref, cds_ref, cdd_ref, w1_ref, b1_ref, w2_ref,
             b2_ref, osrc_ref, odst_ref):
        w1 = w1_ref[...]
        b1 = b1_ref[...]
        w2 = w2_ref[...]
        b2 = b2_ref[...]

        def enc(c_ref0, c_ref1, out_ref, j):
            c0 = c_ref0[pl.ds(j, 1), :]
            c1 = c_ref1[pl.ds(j, 1), :]
            h = (jnp.maximum(c0 * w1 + b1, 0.0)
                 + jnp.maximum(c1 * w1 + b1, 0.0))
            out_ref[j] = jnp.dot(w2, h, preferred_element_type=jnp.float32) \
                + 2.0 * b2

        for j in range(LB):
            enc(css_ref, csd_ref, osrc_ref, j)
            enc(cds_ref, cdd_ref, odst_ref, j)

    cnt_spec = pl.BlockSpec((LB, B), lambda i: (i, 0))
    wcol_spec = pl.BlockSpec((D, 1), lambda i: (0, 0))
    w2_spec = pl.BlockSpec((D, D), lambda i: (0, 0))
    out_spec = pl.BlockSpec((LB, D, B), lambda i: (i, 0, 0))

    return pl.pallas_call(
        body,
        grid=grid,
        in_specs=[cnt_spec, cnt_spec, cnt_spec, cnt_spec,
                  wcol_spec, wcol_spec, w2_spec, wcol_spec],
        out_specs=[out_spec, out_spec],
        out_shape=[jax.ShapeDtypeStruct((L, D, B), jnp.float32)] * 2,
    )(css, csd, cds, cdd, w1c, b1c, w2, b2c)


@jax.jit
def kernel(src_padded_nodes_neighbor_ids, dst_padded_nodes_neighbor_ids,
           W1, b1, W2, b2):
    src = src_padded_nodes_neighbor_ids
    dst = dst_padded_nodes_neighbor_ids

    c_ss, c_sd, c_ds, c_dd = _sc_counts(src, dst)

    b1c = b1.reshape(D, 1)
    b2c = b2.reshape(D, 1)

    out_src_t, out_dst_t = _tc_encode(
        c_ss.reshape(B, L).T, c_sd.reshape(B, L).T,
        c_ds.reshape(B, L).T, c_dd.reshape(B, L).T,
        W1, b1c, W2, b2c)

    out_src = jnp.transpose(out_src_t, (2, 0, 1))
    out_dst = jnp.transpose(out_dst_t, (2, 0, 1))
    return out_src, out_dst


# speedup vs baseline: 2.6487x; 1.4483x over previous
import functools
import jax
import jax.numpy as jnp
from jax import lax
from jax.experimental import pallas as pl
from jax.experimental.pallas import tpu as pltpu
from jax.experimental.pallas import tpu_sc as plsc

D = 64
B = 1024
L = 200
VOCAB = 100000

LP = 256
NCHUNK = LP // 16
HIST_N = 100096
NWORKERS = 32
ROWS_PER_W = B // NWORKERS


def _sc_counts(src, dst):
    mesh = plsc.VectorSubcoreMesh(core_axis_name="c", subcore_axis_name="s")
    f32 = jnp.float32
    RB = 8
    NBLK = ROWS_PER_W // RB

    @functools.partial(
        pl.kernel,
        mesh=mesh,
        compiler_params=pltpu.CompilerParams(needs_layout_passes=False),
        out_type=[jax.ShapeDtypeStruct((B, LP), f32) for _ in range(4)],
        scratch_types=[
            pltpu.VMEM((HIST_N,), f32),
            pltpu.VMEM((2 * RB, 128), jnp.int32),
            pltpu.VMEM((2 * RB, 128), jnp.int32),
            pltpu.VMEM((2 * RB, 128), f32),
            pltpu.VMEM((2 * RB, 128), f32),
            pltpu.VMEM((2 * RB, 128), f32),
            pltpu.VMEM((2 * RB, 128), f32),
        ],
    )
    def sc_kernel(src_hbm, dst_hbm, css_hbm, csd_hbm, cds_hbm, cdd_hbm,
                  hist, sblk, dblk, css, csd, cds, cdd):
        cid = lax.axis_index("c")
        sid = lax.axis_index("s")
        wid = sid * 2 + cid
        base = wid * ROWS_PER_W

        zero16 = jnp.zeros((16,), f32)
        one16 = jnp.ones((16,), f32)

        def zinit(i, carry):
            off = pl.multiple_of(i * 256, 256)
            for j in range(16):
                hist[pl.ds(off + j * 16, 16)] = zero16
            return carry
        lax.fori_loop(0, HIST_N // 256, zinit, 0)

        def chunk(buf, i, c):
            if c < 8:
                return buf.at[i, pl.ds(c * 16, 16)]
            return buf.at[RB + i, pl.ds((c - 8) * 16, 16)]

        def blk_body(r, carry):
            row0 = base + r * RB
            pltpu.sync_copy(src_hbm.at[pl.ds(row0, RB), pl.ds(0, 128)],
                            sblk.at[pl.ds(0, RB), :])
            pltpu.sync_copy(src_hbm.at[pl.ds(row0, RB), pl.ds(128, 128)],
                            sblk.at[pl.ds(RB, RB), :])
            pltpu.sync_copy(dst_hbm.at[pl.ds(row0, RB), pl.ds(0, 128)],
                            dblk.at[pl.ds(0, RB), :])
            pltpu.sync_copy(dst_hbm.at[pl.ds(row0, RB), pl.ds(128, 128)],
                            dblk.at[pl.ds(RB, RB), :])

            for i in range(RB):
                def hist_add(idblk):
                    for c in range(NCHUNK):
                        plsc.addupdate_scatter(hist, [chunk(idblk, i, c)[...]],
                                               one16)

                def hist_clear(idblk):
                    for c in range(NCHUNK):
                        plsc.store_scatter(hist, [chunk(idblk, i, c)[...]],
                                           zero16)

                def gather_counts(idblk, out_blk):
                    for c in range(NCHUNK):
                        idx = chunk(idblk, i, c)[...]
                        g = plsc.load_gather(hist, [idx])
                        chunk(out_blk, i, c)[...] = jnp.where(idx == 0, 0.0, g)

                hist_add(sblk)
                gather_counts(sblk, css)
                gather_counts(dblk, cds)
                hist_clear(sblk)

                hist_add(dblk)
                gather_counts(sblk, csd)
                gather_counts(dblk, cdd)
                hist_clear(dblk)

            for cbuf, chbm in ((css, css_hbm), (csd, csd_hbm),
                               (cds, cds_hbm), (cdd, cdd_hbm)):
                pltpu.sync_copy(cbuf.at[pl.ds(0, RB), :],
                                chbm.at[pl.ds(row0, RB), pl.ds(0, 128)])
                pltpu.sync_copy(cbuf.at[pl.ds(RB, RB), :],
                                chbm.at[pl.ds(row0, RB), pl.ds(128, 128)])
            return carry

        lax.fori_loop(0, NBLK, blk_body, 0)

    srcp, dstp = _tc_pad(src, dst)
    return sc_kernel(srcp, dstp)


def _tc_pad(src, dst):
    PB = 256

    def body(s_ref, d_ref, so_ref, do_ref):
        zeros = jnp.zeros((PB, LP - L), jnp.int32)
        so_ref[:, :L] = s_ref[...]
        so_ref[:, L:] = zeros
        do_ref[:, :L] = d_ref[...]
        do_ref[:, L:] = zeros

    in_spec = pl.BlockSpec((PB, L), lambda i: (i, 0))
    out_spec = pl.BlockSpec((PB, LP), lambda i: (i, 0))
    return pl.pallas_call(
        body,
        grid=(B // PB,),
        in_specs=[in_spec, in_spec],
        out_specs=[out_spec, out_spec],
        out_shape=[jax.ShapeDtypeStruct((B, LP), jnp.int32)] * 2,
    )(src, dst)


def _tc_encode(css, csd, cds, cdd, w1c, b1c, w2, b2c):
    LB = 8
    grid = (L // LB,)

    def body(css_ref, csd_ref, cds_ref, cdd_ref, w1_ref, b1_ref, w2_ref,
             b2_ref, osrc_ref, odst_ref):
        w1 = w1_ref[...]
        b1 = b1_ref[...]
        w2 = w2_ref[...]
        b2 = b2_ref[...]

        def enc(c_ref0, c_ref1, out_ref, j):
            c0 = c_ref0[pl.ds(j, 1), :]
            c1 = c_ref1[pl.ds(j, 1), :]
            h = (jnp.maximum(c0 * w1 + b1, 0.0)
                 + jnp.maximum(c1 * w1 + b1, 0.0))
            out_ref[j] = jnp.dot(w2, h, preferred_element_type=jnp.float32) \
                + 2.0 * b2

        for j in range(LB):
            enc(css_ref, csd_ref, osrc_ref, j)
            enc(cds_ref, cdd_ref, odst_ref, j)

    cnt_spec = pl.BlockSpec((LB, B), lambda i: (i, 0))
    wcol_spec = pl.BlockSpec((D, 1), lambda i: (0, 0))
    w2_spec = pl.BlockSpec((D, D), lambda i: (0, 0))
    out_spec = pl.BlockSpec((LB, D, B), lambda i: (i, 0, 0))

    return pl.pallas_call(
        body,
        grid=grid,
        in_specs=[cnt_spec, cnt_spec, cnt_spec, cnt_spec,
                  wcol_spec, wcol_spec, w2_spec, wcol_spec],
        out_specs=[out_spec, out_spec],
        out_shape=[jax.ShapeDtypeStruct((L, D, B), jnp.float32)] * 2,
    )(css, csd, cds, cdd, w1c, b1c, w2, b2c)


@jax.jit
def kernel(src_padded_nodes_neighbor_ids, dst_padded_nodes_neighbor_ids,
           W1, b1, W2, b2):
    src = src_padded_nodes_neighbor_ids
    dst = dst_padded_nodes_neighbor_ids

    c_ss, c_sd, c_ds, c_dd = _sc_counts(src, dst)

    b1c = b1.reshape(D, 1)
    b2c = b2.reshape(D, 1)

    out_src_t, out_dst_t = _tc_encode(
        c_ss[:, :L].T, c_sd[:, :L].T,
        c_ds[:, :L].T, c_dd[:, :L].T,
        W1, b1c, W2, b2c)

    out_src = jnp.transpose(out_src_t, (2, 0, 1))
    out_dst = jnp.transpose(out_dst_t, (2, 0, 1))
    return out_src, out_dst
